# baseline (speedup 1.0000x reference)
import jax
import jax.numpy as jnp
from jax.experimental import pallas as pl


def _copy_block(x_ref, o_ref):
    o_ref[...] = x_ref[...]


def kernel(X):
    b, n, f = X.shape
    chunk = 320
    grid = (b, n // chunk)
    return pl.pallas_call(
        _copy_block,
        grid=grid,
        in_specs=[pl.BlockSpec((1, chunk, f), lambda i, j: (i, j, 0))],
        out_specs=pl.BlockSpec((1, chunk, f), lambda i, j: (i, j, 0)),
        out_shape=jax.ShapeDtypeStruct((b, n, f), X.dtype),
    )(X)
